# baseline (speedup 1.0000x reference)
import functools

import jax
import jax.numpy as jnp
from jax import lax
from jax.experimental import pallas as pl
from jax.experimental.pallas import tpu as pltpu
from jax.experimental.pallas import tpu_sc as plsc

_L = 25
_R = 5
_PH = 49408
_D = 768
_SCALE = 1.0
_LMAX = 13


def _sc_body(nc, tok_hbm, emb_hbm, upf_hbm, down_hbm, bias_hbm, out_hbm,
             tok_v, emb_v, up_v, down_v, bias_v, pe_v,
             read_sem, big_sem, small_sem):
    cid = lax.axis_index("c")
    sid = lax.axis_index("s")
    wid = sid * nc + cid
    b = wid // 2
    half = wid % 2
    l_lo = half * _LMAX
    l_hi = jnp.where(half == 0, _LMAX, _L)
    n_seq = emb_v.shape[0]

    pltpu.make_async_copy(tok_hbm, tok_v, read_sem).start()
    pltpu.make_async_copy(emb_hbm.at[b], emb_v, read_sem).start()
    pltpu.make_async_copy(upf_hbm, up_v, read_sem).start()
    pltpu.make_async_copy(down_hbm, down_v, read_sem).start()
    pltpu.make_async_copy(bias_hbm, bias_v, read_sem).start()
    pltpu.make_async_copy(tok_hbm, tok_v, read_sem).wait()
    pltpu.make_async_copy(emb_hbm.at[b], emb_v, read_sem).wait()
    pltpu.make_async_copy(upf_hbm, up_v, read_sem).wait()
    pltpu.make_async_copy(down_hbm, down_v, read_sem).wait()
    pltpu.make_async_copy(bias_hbm, bias_v, read_sem).wait()

    for l_off in range(_LMAX):
        l = l_lo + l_off

        @pl.when(l < l_hi)
        def _():
            pltpu.make_async_copy(emb_v, out_hbm.at[b * _L + l], big_sem).start()

    pos = jnp.int32(-1)
    n_tok = tok_v.shape[1]
    starts = list(range(0, n_tok - 16, 16)) + [n_tok - 16]
    for s in starts:
        chunk = tok_v[b, pl.ds(s, 16)]
        for j in range(16):
            pos = jnp.where(chunk[j] == _PH, s + j, pos)
    row = jnp.clip(pos, 0, n_seq - 1)

    n_up = up_v.shape[0]
    up_starts = list(range(0, n_up - 16, 16)) + [n_up - 16]
    up_chunks = [up_v[pl.ds(s, 16)] for s in up_starts]

    def up_scalar(idx):
        if idx >= up_starts[-1]:
            return up_chunks[-1][idx - up_starts[-1]]
        return up_chunks[idx // 16][idx % 16]

    for j in range(_D // 16):
        sl = pl.ds(j * 16, 16)
        bj = bias_v[sl]
        dr = [down_v[r, sl] for r in range(_R)]
        for l in range(_L):
            acc = bj
            for r in range(_R):
                acc = acc + up_scalar(_R * l + r) * dr[r]
            pe_v[l, sl] = acc

    for l_off in range(_LMAX):
        l = l_lo + l_off

        @pl.when(l < l_hi)
        def _():
            pltpu.make_async_copy(emb_v, out_hbm.at[b * _L + l], big_sem).wait()

    for l_off in range(_LMAX):
        l = l_lo + l_off

        @pl.when((l < l_hi) & (pos >= 0))
        def _():
            pltpu.make_async_copy(
                pe_v.at[l], out_hbm.at[b * _L + l, row], small_sem).start()

    for l_off in range(_LMAX):
        l = l_lo + l_off

        @pl.when((l < l_hi) & (pos >= 0))
        def _():
            pltpu.make_async_copy(
                pe_v.at[l], out_hbm.at[b * _L + l, row], small_sem).wait()


def kernel(tokenized_text, embedded_text, lora_up, lora_down, bias):
    b_dim, n = tokenized_text.shape
    up_flat = lora_up.reshape(-1)

    info = plsc.get_sparse_core_info()
    nc = info.num_cores
    mesh = plsc.VectorSubcoreMesh(core_axis_name="c", subcore_axis_name="s")

    out = pl.kernel(
        functools.partial(_sc_body, nc),
        out_type=jax.ShapeDtypeStruct((b_dim * _L, n, _D), jnp.float32),
        mesh=mesh,
        scratch_types=[
            pltpu.VMEM((b_dim, n), jnp.int32),
            pltpu.VMEM((n, _D), jnp.float32),
            pltpu.VMEM((_L * _R,), jnp.float32),
            pltpu.VMEM((_R, _D), jnp.float32),
            pltpu.VMEM((_D,), jnp.float32),
            pltpu.VMEM((_L, _D), jnp.float32),
            pltpu.SemaphoreType.DMA,
            pltpu.SemaphoreType.DMA,
            pltpu.SemaphoreType.DMA,
        ],
    )(tokenized_text, embedded_text, up_flat, lora_down, bias)
    return out
